# Initial kernel scaffold; baseline (speedup 1.0000x reference)
import jax
import jax.numpy as jnp
from jax.experimental import pallas as pl


def kernel(x1, x2, encoder, encoder_vel, W_ih, W_hh, b_ih, b_hh, W_dec, b_dec, W_dec_vel, b_dec_vel):
    raise NotImplementedError("write your pallas kernel here")


# speedup vs baseline: 1.5785x; 1.5785x over previous
import functools

import jax
import jax.numpy as jnp
from jax import lax
from jax.experimental import pallas as pl
from jax.experimental.pallas import tpu as pltpu
from jax.experimental.pallas import tpu_sc as plsc

_VOCAB = 1000
_VVOCAB = 128
_EMB = 64
_HID = 512
_B = 1024
_T = 20
_BBLK = 128
_CHUNK = 128
_EMBP = 128
_WAVE = 5


def _sc_gather_rows(table, idx):
    info = plsc.get_sparse_core_info()
    nc, ns = info.num_cores, info.num_subcores
    nw = nc * ns
    k = idx.shape[1]
    mesh = plsc.VectorSubcoreMesh(core_axis_name="c", subcore_axis_name="s")

    @functools.partial(
        pl.kernel,
        mesh=mesh,
        out_type=jax.ShapeDtypeStruct((nw, k, _CHUNK, _EMBP), jnp.float32),
        scratch_types=[
            pltpu.VMEM((k, _CHUNK), jnp.int32),
            pltpu.VMEM((_WAVE, _CHUNK, _EMBP), jnp.float32),
            pltpu.SemaphoreType.DMA,
        ],
    )
    def run(table_hbm, idx_hbm, out_hbm, idx_v, rows_v, sem):
        wid = lax.axis_index("s") * nc + lax.axis_index("c")
        pltpu.sync_copy(idx_hbm.at[wid], idx_v)
        for w in range(k // _WAVE):
            cps = [
                pltpu.async_copy(
                    table_hbm.at[idx_v.at[w * _WAVE + j]], rows_v.at[j], sem)
                for j in range(_WAVE)
            ]
            for cp in cps:
                cp.wait()
            pltpu.sync_copy(rows_v, out_hbm.at[wid].at[pl.ds(w * _WAVE, _WAVE)])

    return run(table, idx)


def _fused_body(g_ref, wih_ref, whh_ref, bg_ref, wd_ref, bd_ref, wv_ref, bv_ref,
                out_ref, outv_ref, ht_ref, ct_ref, h_sc, c_sc):
    t = pl.program_id(1)

    @pl.when(t == 0)
    def _():
        h_sc[...] = jnp.zeros_like(h_sc)
        c_sc[...] = jnp.zeros_like(c_sc)

    x = g_ref[0, 0] + g_ref[1, 0]
    h = h_sc[...]
    c = c_sc[...]
    gates = (jnp.dot(x, wih_ref[...], preferred_element_type=jnp.float32)
             + jnp.dot(h, whh_ref[...], preferred_element_type=jnp.float32)
             + bg_ref[...])
    gi = jax.nn.sigmoid(gates[:, :_HID])
    gf = jax.nn.sigmoid(gates[:, _HID:2 * _HID])
    gg = jnp.tanh(gates[:, 2 * _HID:3 * _HID])
    go = jax.nn.sigmoid(gates[:, 3 * _HID:])
    c2 = gf * c + gi * gg
    h2 = go * jnp.tanh(c2)
    h_sc[...] = h2
    c_sc[...] = c2

    logits = jnp.dot(h2, wd_ref[...], preferred_element_type=jnp.float32) + bd_ref[...]
    sh = logits - jnp.max(logits, axis=-1, keepdims=True)
    lse = jnp.log(jnp.sum(jnp.exp(sh), axis=-1, keepdims=True))
    out_ref[:, pl.ds(t, 1), :] = (sh - lse)[:, None, :]

    vlog = jnp.dot(h2, wv_ref[...], preferred_element_type=jnp.float32) + bv_ref[...]
    vsh = vlog - jnp.max(vlog, axis=-1, keepdims=True)
    vlse = jnp.log(jnp.sum(jnp.exp(vsh), axis=-1, keepdims=True))
    outv_ref[:, pl.ds(t, 1), :] = (vsh - vlse)[:, None, :]

    ht_ref[0] = h2
    ct_ref[0] = c2


def _tc_call(g, wihT, whhT, bg, wdT, bd, wvT, bv):
    nb = _B // _BBLK
    return pl.pallas_call(
        _fused_body,
        grid=(nb, _T),
        in_specs=[
            pl.BlockSpec((2, 1, _BBLK, _EMBP), lambda i, t: (0, t, i, 0)),
            pl.BlockSpec((_EMBP, 4 * _HID), lambda i, t: (0, 0)),
            pl.BlockSpec((_HID, 4 * _HID), lambda i, t: (0, 0)),
            pl.BlockSpec((1, 4 * _HID), lambda i, t: (0, 0)),
            pl.BlockSpec((_HID, _VOCAB), lambda i, t: (0, 0)),
            pl.BlockSpec((1, _VOCAB), lambda i, t: (0, 0)),
            pl.BlockSpec((_HID, _VVOCAB), lambda i, t: (0, 0)),
            pl.BlockSpec((1, _VVOCAB), lambda i, t: (0, 0)),
        ],
        out_specs=[
            pl.BlockSpec((_BBLK, _T, _VOCAB), lambda i, t: (i, 0, 0)),
            pl.BlockSpec((_BBLK, _T, _VVOCAB), lambda i, t: (i, 0, 0)),
            pl.BlockSpec((1, _BBLK, _HID), lambda i, t: (0, i, 0)),
            pl.BlockSpec((1, _BBLK, _HID), lambda i, t: (0, i, 0)),
        ],
        out_shape=[
            jax.ShapeDtypeStruct((_B, _T, _VOCAB), jnp.float32),
            jax.ShapeDtypeStruct((_B, _T, _VVOCAB), jnp.float32),
            jax.ShapeDtypeStruct((1, _B, _HID), jnp.float32),
            jax.ShapeDtypeStruct((1, _B, _HID), jnp.float32),
        ],
        scratch_shapes=[
            pltpu.VMEM((_BBLK, _HID), jnp.float32),
            pltpu.VMEM((_BBLK, _HID), jnp.float32),
        ],
        compiler_params=pltpu.CompilerParams(
            dimension_semantics=("arbitrary", "arbitrary"),
            vmem_limit_bytes=100 * 1024 * 1024,
        ),
    )(g, wihT, whhT, bg, wdT, bd, wvT, bv)


def kernel(x1, x2, encoder, encoder_vel, W_ih, W_hh, b_ih, b_hh, W_dec, b_dec,
           W_dec_vel, b_dec_vel):
    info = plsc.get_sparse_core_info()
    nw = info.num_cores * info.num_subcores
    idx = jnp.concatenate([
        jnp.swapaxes(x1, 0, 1).reshape(-1),
        jnp.swapaxes(x2, 0, 1).reshape(-1),
    ]).astype(jnp.int32)
    idx3 = idx.reshape(nw, -1, _CHUNK)
    table_p = jnp.pad(encoder, ((0, 0), (0, _EMBP - _EMB)))
    rows = _sc_gather_rows(table_p, idx3)
    g = rows.reshape(2, _T, _B, _EMBP)

    bg = (b_ih + b_hh).reshape(1, 4 * _HID)
    wihT_p = jnp.pad(W_ih.T, ((0, _EMBP - _EMB), (0, 0)))
    out, outv, ht, ct = _tc_call(
        g, wihT_p, W_hh.T, bg, W_dec.T, b_dec.reshape(1, _VOCAB),
        W_dec_vel.T, b_dec_vel.reshape(1, _VVOCAB))
    return (out, outv, (ht, ct))


# speedup vs baseline: 1.5795x; 1.0006x over previous
import functools

import jax
import jax.numpy as jnp
from jax import lax
from jax.experimental import pallas as pl
from jax.experimental.pallas import tpu as pltpu
from jax.experimental.pallas import tpu_sc as plsc

_VOCAB = 1000
_VVOCAB = 128
_EMB = 64
_HID = 512
_B = 1024
_T = 20
_BBLK = 128
_CHUNK = 128
_EMBP = 128
_WAVE = 5


def _sc_gather_rows(table, idx):
    info = plsc.get_sparse_core_info()
    nc, ns = info.num_cores, info.num_subcores
    nw = nc * ns
    k = idx.shape[1]
    mesh = plsc.VectorSubcoreMesh(core_axis_name="c", subcore_axis_name="s")

    @functools.partial(
        pl.kernel,
        mesh=mesh,
        out_type=jax.ShapeDtypeStruct((nw, k, _CHUNK, _EMBP), jnp.float32),
        scratch_types=[
            pltpu.VMEM((k, _CHUNK), jnp.int32),
            pltpu.VMEM((_WAVE, _CHUNK, _EMBP), jnp.float32),
            pltpu.SemaphoreType.DMA,
        ],
    )
    def run(table_hbm, idx_hbm, out_hbm, idx_v, rows_v, sem):
        wid = lax.axis_index("s") * nc + lax.axis_index("c")
        pltpu.sync_copy(idx_hbm.at[wid], idx_v)
        for w in range(k // _WAVE):
            cps = [
                pltpu.async_copy(
                    table_hbm.at[idx_v.at[w * _WAVE + j]], rows_v.at[j], sem)
                for j in range(_WAVE)
            ]
            for cp in cps:
                cp.wait()
            pltpu.sync_copy(rows_v, out_hbm.at[wid].at[pl.ds(w * _WAVE, _WAVE)])

    return run(table, idx)


def _fused_body(g_ref, wih_ref, whh_ref, bg_ref, wd_ref, bd_ref, wv_ref, bv_ref,
                out_ref, outv_ref, ht_ref, ct_ref, h_sc, c_sc):
    t = pl.program_id(1)

    @pl.when(t == 0)
    def _():
        h_sc[...] = jnp.zeros_like(h_sc)
        c_sc[...] = jnp.zeros_like(c_sc)

    x = (g_ref[0, 0] + g_ref[1, 0]).astype(jnp.bfloat16)
    h = h_sc[...]
    c = c_sc[...]
    gates = (jnp.dot(x, wih_ref[...], preferred_element_type=jnp.float32)
             + jnp.dot(h.astype(jnp.bfloat16), whh_ref[...],
                       preferred_element_type=jnp.float32)
             + bg_ref[...])
    gi = jax.nn.sigmoid(gates[:, :_HID])
    gf = jax.nn.sigmoid(gates[:, _HID:2 * _HID])
    gg = jnp.tanh(gates[:, 2 * _HID:3 * _HID])
    go = jax.nn.sigmoid(gates[:, 3 * _HID:])
    c2 = gf * c + gi * gg
    h2 = go * jnp.tanh(c2)
    h_sc[...] = h2
    c_sc[...] = c2
    h2b = h2.astype(jnp.bfloat16)

    logits = jnp.dot(h2b, wd_ref[...], preferred_element_type=jnp.float32) + bd_ref[...]
    sh = logits - jnp.max(logits, axis=-1, keepdims=True)
    lse = jnp.log(jnp.sum(jnp.exp(sh), axis=-1, keepdims=True))
    out_ref[:, pl.ds(t, 1), :] = (sh - lse)[:, None, :]

    vlog = jnp.dot(h2b, wv_ref[...], preferred_element_type=jnp.float32) + bv_ref[...]
    vsh = vlog - jnp.max(vlog, axis=-1, keepdims=True)
    vlse = jnp.log(jnp.sum(jnp.exp(vsh), axis=-1, keepdims=True))
    outv_ref[:, pl.ds(t, 1), :] = (vsh - vlse)[:, None, :]

    ht_ref[0] = h2
    ct_ref[0] = c2


def _tc_call(g, wihT, whhT, bg, wdT, bd, wvT, bv):
    nb = _B // _BBLK
    return pl.pallas_call(
        _fused_body,
        grid=(nb, _T),
        in_specs=[
            pl.BlockSpec((2, 1, _BBLK, _EMBP), lambda i, t: (0, t, i, 0)),
            pl.BlockSpec((_EMBP, 4 * _HID), lambda i, t: (0, 0)),
            pl.BlockSpec((_HID, 4 * _HID), lambda i, t: (0, 0)),
            pl.BlockSpec((1, 4 * _HID), lambda i, t: (0, 0)),
            pl.BlockSpec((_HID, _VOCAB), lambda i, t: (0, 0)),
            pl.BlockSpec((1, _VOCAB), lambda i, t: (0, 0)),
            pl.BlockSpec((_HID, _VVOCAB), lambda i, t: (0, 0)),
            pl.BlockSpec((1, _VVOCAB), lambda i, t: (0, 0)),
        ],
        out_specs=[
            pl.BlockSpec((_BBLK, _T, _VOCAB), lambda i, t: (i, 0, 0)),
            pl.BlockSpec((_BBLK, _T, _VVOCAB), lambda i, t: (i, 0, 0)),
            pl.BlockSpec((1, _BBLK, _HID), lambda i, t: (0, i, 0)),
            pl.BlockSpec((1, _BBLK, _HID), lambda i, t: (0, i, 0)),
        ],
        out_shape=[
            jax.ShapeDtypeStruct((_B, _T, _VOCAB), jnp.float32),
            jax.ShapeDtypeStruct((_B, _T, _VVOCAB), jnp.float32),
            jax.ShapeDtypeStruct((1, _B, _HID), jnp.float32),
            jax.ShapeDtypeStruct((1, _B, _HID), jnp.float32),
        ],
        scratch_shapes=[
            pltpu.VMEM((_BBLK, _HID), jnp.float32),
            pltpu.VMEM((_BBLK, _HID), jnp.float32),
        ],
        compiler_params=pltpu.CompilerParams(
            dimension_semantics=("arbitrary", "arbitrary"),
            vmem_limit_bytes=100 * 1024 * 1024,
        ),
    )(g, wihT, whhT, bg, wdT, bd, wvT, bv)


def kernel(x1, x2, encoder, encoder_vel, W_ih, W_hh, b_ih, b_hh, W_dec, b_dec,
           W_dec_vel, b_dec_vel):
    info = plsc.get_sparse_core_info()
    nw = info.num_cores * info.num_subcores
    idx = jnp.concatenate([
        jnp.swapaxes(x1, 0, 1).reshape(-1),
        jnp.swapaxes(x2, 0, 1).reshape(-1),
    ]).astype(jnp.int32)
    idx3 = idx.reshape(nw, -1, _CHUNK)
    table_p = jnp.pad(encoder, ((0, 0), (0, _EMBP - _EMB)))
    rows = _sc_gather_rows(table_p, idx3)
    g = rows.reshape(2, _T, _B, _EMBP)

    bg = (b_ih + b_hh).reshape(1, 4 * _HID)
    bf16 = jnp.bfloat16
    wihT_p = jnp.pad(W_ih.T, ((0, _EMBP - _EMB), (0, 0))).astype(bf16)
    out, outv, ht, ct = _tc_call(
        g, wihT_p, W_hh.T.astype(bf16), bg, W_dec.T.astype(bf16),
        b_dec.reshape(1, _VOCAB), W_dec_vel.T.astype(bf16),
        b_dec_vel.reshape(1, _VVOCAB))
    return (out, outv, (ht, ct))


# speedup vs baseline: 1.7597x; 1.1141x over previous
import functools

import jax
import jax.numpy as jnp
from jax import lax
from jax.experimental import pallas as pl
from jax.experimental.pallas import tpu as pltpu
from jax.experimental.pallas import tpu_sc as plsc

_VOCAB = 1000
_VVOCAB = 128
_EMB = 64
_HID = 512
_B = 1024
_T = 20
_BBLK = 256
_CHUNK = 128
_EMBP = 128
_WAVE = 5


def _sc_gather_rows(table, idx):
    info = plsc.get_sparse_core_info()
    nc, ns = info.num_cores, info.num_subcores
    nw = nc * ns
    k = idx.shape[1]
    mesh = plsc.VectorSubcoreMesh(core_axis_name="c", subcore_axis_name="s")

    @functools.partial(
        pl.kernel,
        mesh=mesh,
        out_type=jax.ShapeDtypeStruct((nw, k, _CHUNK, _EMBP), jnp.float32),
        scratch_types=[
            pltpu.VMEM((k, _CHUNK), jnp.int32),
            pltpu.VMEM((_WAVE, _CHUNK, _EMBP), jnp.float32),
            pltpu.SemaphoreType.DMA,
        ],
    )
    def run(table_hbm, idx_hbm, out_hbm, idx_v, rows_v, sem):
        wid = lax.axis_index("s") * nc + lax.axis_index("c")
        pltpu.sync_copy(idx_hbm.at[wid], idx_v)
        for w in range(k // _WAVE):
            cps = [
                pltpu.async_copy(
                    table_hbm.at[idx_v.at[w * _WAVE + j]], rows_v.at[j], sem)
                for j in range(_WAVE)
            ]
            for cp in cps:
                cp.wait()
            pltpu.sync_copy(rows_v, out_hbm.at[wid].at[pl.ds(w * _WAVE, _WAVE)])

    return run(table, idx)


def _fused_body(g_ref, wih_ref, whh_ref, bg_ref, wd_ref, bd_ref, wv_ref, bv_ref,
                out_ref, outv_ref, ht_ref, ct_ref, h_sc, c_sc):
    t = pl.program_id(1)

    @pl.when(t == 0)
    def _():
        h_sc[...] = jnp.zeros_like(h_sc)
        c_sc[...] = jnp.zeros_like(c_sc)

    x = (g_ref[0, 0] + g_ref[1, 0]).astype(jnp.bfloat16)
    h = h_sc[...]
    c = c_sc[...]
    gates = (jnp.dot(x, wih_ref[...], preferred_element_type=jnp.float32)
             + jnp.dot(h.astype(jnp.bfloat16), whh_ref[...],
                       preferred_element_type=jnp.float32)
             + bg_ref[...])
    gi = jax.nn.sigmoid(gates[:, :_HID])
    gf = jax.nn.sigmoid(gates[:, _HID:2 * _HID])
    gg = jnp.tanh(gates[:, 2 * _HID:3 * _HID])
    go = jax.nn.sigmoid(gates[:, 3 * _HID:])
    c2 = gf * c + gi * gg
    h2 = go * jnp.tanh(c2)
    h_sc[...] = h2
    c_sc[...] = c2
    h2b = h2.astype(jnp.bfloat16)

    logits = jnp.dot(h2b, wd_ref[...], preferred_element_type=jnp.float32) + bd_ref[...]
    sh = logits - jnp.max(logits, axis=-1, keepdims=True)
    lse = jnp.log(jnp.sum(jnp.exp(sh), axis=-1, keepdims=True))
    out_ref[:, pl.ds(t, 1), :] = (sh - lse)[:, None, :]

    vlog = jnp.dot(h2b, wv_ref[...], preferred_element_type=jnp.float32) + bv_ref[...]
    vsh = vlog - jnp.max(vlog, axis=-1, keepdims=True)
    vlse = jnp.log(jnp.sum(jnp.exp(vsh), axis=-1, keepdims=True))
    outv_ref[:, pl.ds(t, 1), :] = (vsh - vlse)[:, None, :]

    ht_ref[0] = h2
    ct_ref[0] = c2


def _tc_call(g, wihT, whhT, bg, wdT, bd, wvT, bv):
    nb = _B // _BBLK
    return pl.pallas_call(
        _fused_body,
        grid=(nb, _T),
        in_specs=[
            pl.BlockSpec((2, 1, _BBLK, _EMBP), lambda i, t: (0, t, i, 0)),
            pl.BlockSpec((_EMBP, 4 * _HID), lambda i, t: (0, 0)),
            pl.BlockSpec((_HID, 4 * _HID), lambda i, t: (0, 0)),
            pl.BlockSpec((1, 4 * _HID), lambda i, t: (0, 0)),
            pl.BlockSpec((_HID, _VOCAB), lambda i, t: (0, 0)),
            pl.BlockSpec((1, _VOCAB), lambda i, t: (0, 0)),
            pl.BlockSpec((_HID, _VVOCAB), lambda i, t: (0, 0)),
            pl.BlockSpec((1, _VVOCAB), lambda i, t: (0, 0)),
        ],
        out_specs=[
            pl.BlockSpec((_BBLK, _T, _VOCAB), lambda i, t: (i, 0, 0)),
            pl.BlockSpec((_BBLK, _T, _VVOCAB), lambda i, t: (i, 0, 0)),
            pl.BlockSpec((1, _BBLK, _HID), lambda i, t: (0, i, 0)),
            pl.BlockSpec((1, _BBLK, _HID), lambda i, t: (0, i, 0)),
        ],
        out_shape=[
            jax.ShapeDtypeStruct((_B, _T, _VOCAB), jnp.float32),
            jax.ShapeDtypeStruct((_B, _T, _VVOCAB), jnp.float32),
            jax.ShapeDtypeStruct((1, _B, _HID), jnp.float32),
            jax.ShapeDtypeStruct((1, _B, _HID), jnp.float32),
        ],
        scratch_shapes=[
            pltpu.VMEM((_BBLK, _HID), jnp.float32),
            pltpu.VMEM((_BBLK, _HID), jnp.float32),
        ],
        compiler_params=pltpu.CompilerParams(
            dimension_semantics=("arbitrary", "arbitrary"),
            vmem_limit_bytes=100 * 1024 * 1024,
        ),
    )(g, wihT, whhT, bg, wdT, bd, wvT, bv)


def kernel(x1, x2, encoder, encoder_vel, W_ih, W_hh, b_ih, b_hh, W_dec, b_dec,
           W_dec_vel, b_dec_vel):
    info = plsc.get_sparse_core_info()
    nw = info.num_cores * info.num_subcores
    idx = jnp.concatenate([
        jnp.swapaxes(x1, 0, 1).reshape(-1),
        jnp.swapaxes(x2, 0, 1).reshape(-1),
    ]).astype(jnp.int32)
    idx3 = idx.reshape(nw, -1, _CHUNK)
    table_p = jnp.pad(encoder, ((0, 0), (0, _EMBP - _EMB)))
    rows = _sc_gather_rows(table_p, idx3)
    g = rows.reshape(2, _T, _B, _EMBP)

    bg = (b_ih + b_hh).reshape(1, 4 * _HID)
    bf16 = jnp.bfloat16
    wihT_p = jnp.pad(W_ih.T, ((0, _EMBP - _EMB), (0, 0))).astype(bf16)
    out, outv, ht, ct = _tc_call(
        g, wihT_p, W_hh.T.astype(bf16), bg, W_dec.T.astype(bf16),
        b_dec.reshape(1, _VOCAB), W_dec_vel.T.astype(bf16),
        b_dec_vel.reshape(1, _VVOCAB))
    return (out, outv, (ht, ct))


# speedup vs baseline: 1.7739x; 1.0081x over previous
import functools

import jax
import jax.numpy as jnp
from jax import lax
from jax.experimental import pallas as pl
from jax.experimental.pallas import tpu as pltpu
from jax.experimental.pallas import tpu_sc as plsc

_VOCAB = 1000
_VVOCAB = 128
_EMB = 64
_HID = 512
_B = 1024
_T = 20
_TP = 24
_DB = 16
_CHUNK = 128
_EMBP = 128
_WAVE = 5


def _sc_gather_rows(table, idx):
    info = plsc.get_sparse_core_info()
    nc, ns = info.num_cores, info.num_subcores
    nw = nc * ns
    k = idx.shape[1]
    mesh = plsc.VectorSubcoreMesh(core_axis_name="c", subcore_axis_name="s")

    @functools.partial(
        pl.kernel,
        mesh=mesh,
        out_type=jax.ShapeDtypeStruct((nw, k, _CHUNK, _EMBP), jnp.float32),
        scratch_types=[
            pltpu.VMEM((k, _CHUNK), jnp.int32),
            pltpu.VMEM((_WAVE, _CHUNK, _EMBP), jnp.float32),
            pltpu.SemaphoreType.DMA,
        ],
    )
    def run(table_hbm, idx_hbm, out_hbm, idx_v, rows_v, sem):
        wid = lax.axis_index("s") * nc + lax.axis_index("c")
        pltpu.sync_copy(idx_hbm.at[wid], idx_v)
        for w in range(k // _WAVE):
            cps = [
                pltpu.async_copy(
                    table_hbm.at[idx_v.at[w * _WAVE + j]], rows_v.at[j], sem)
                for j in range(_WAVE)
            ]
            for cp in cps:
                cp.wait()
            pltpu.sync_copy(rows_v, out_hbm.at[wid].at[pl.ds(w * _WAVE, _WAVE)])

    return run(table, idx)


def _lstm_body(g_ref, wih_ref, whh_ref, bg_ref, hall_ref, ht_ref, ct_ref,
               h_sc, c_sc):
    t = pl.program_id(0)

    @pl.when(t == 0)
    def _():
        h_sc[...] = jnp.zeros_like(h_sc)
        c_sc[...] = jnp.zeros_like(c_sc)

    x = (g_ref[0, 0] + g_ref[1, 0]).astype(jnp.bfloat16)
    h = h_sc[...]
    c = c_sc[...]
    gates = (jnp.dot(x, wih_ref[...], preferred_element_type=jnp.float32)
             + jnp.dot(h.astype(jnp.bfloat16), whh_ref[...],
                       preferred_element_type=jnp.float32)
             + bg_ref[...])
    gi = jax.nn.sigmoid(gates[:, :_HID])
    gf = jax.nn.sigmoid(gates[:, _HID:2 * _HID])
    gg = jnp.tanh(gates[:, 2 * _HID:3 * _HID])
    go = jax.nn.sigmoid(gates[:, 3 * _HID:])
    c2 = gf * c + gi * gg
    h2 = go * jnp.tanh(c2)
    h_sc[...] = h2
    c_sc[...] = c2
    hall_ref[0] = h2.astype(jnp.bfloat16)
    ht_ref[0] = h2
    ct_ref[0] = c2


def _lstm_call(g, wihT, whhT, bg):
    return pl.pallas_call(
        _lstm_body,
        grid=(_T,),
        in_specs=[
            pl.BlockSpec((2, 1, _B, _EMBP), lambda t: (0, t, 0, 0)),
            pl.BlockSpec((_EMBP, 4 * _HID), lambda t: (0, 0)),
            pl.BlockSpec((_HID, 4 * _HID), lambda t: (0, 0)),
            pl.BlockSpec((1, 4 * _HID), lambda t: (0, 0)),
        ],
        out_specs=[
            pl.BlockSpec((1, _B, _HID), lambda t: (t, 0, 0)),
            pl.BlockSpec((1, _B, _HID), lambda t: (0, 0, 0)),
            pl.BlockSpec((1, _B, _HID), lambda t: (0, 0, 0)),
        ],
        out_shape=[
            jax.ShapeDtypeStruct((_T, _B, _HID), jnp.bfloat16),
            jax.ShapeDtypeStruct((1, _B, _HID), jnp.float32),
            jax.ShapeDtypeStruct((1, _B, _HID), jnp.float32),
        ],
        scratch_shapes=[
            pltpu.VMEM((_B, _HID), jnp.float32),
            pltpu.VMEM((_B, _HID), jnp.float32),
        ],
        compiler_params=pltpu.CompilerParams(
            dimension_semantics=("arbitrary",),
            vmem_limit_bytes=100 * 1024 * 1024,
        ),
    )(g, wihT, whhT, bg)


def _dec_body(h_ref, wd_ref, bd_ref, wv_ref, bv_ref, out_ref, outv_ref):
    rows = h_ref[...]
    logits = jnp.dot(rows, wd_ref[...], preferred_element_type=jnp.float32) + bd_ref[...]
    sh = logits - jnp.max(logits, axis=-1, keepdims=True)
    sm = sh - jnp.log(jnp.sum(jnp.exp(sh), axis=-1, keepdims=True))
    vlog = jnp.dot(rows, wv_ref[...], preferred_element_type=jnp.float32) + bv_ref[...]
    vsh = vlog - jnp.max(vlog, axis=-1, keepdims=True)
    vm = vsh - jnp.log(jnp.sum(jnp.exp(vsh), axis=-1, keepdims=True))
    for bb in range(_DB):
        out_ref[bb] = sm[bb * _TP:bb * _TP + _T, :]
        outv_ref[bb] = vm[bb * _TP:bb * _TP + _T, :]


def _dec_call(hrows, wdT, bd, wvT, bv):
    nb = _B // _DB
    return pl.pallas_call(
        _dec_body,
        grid=(nb,),
        in_specs=[
            pl.BlockSpec((_DB * _TP, _HID), lambda i: (i, 0)),
            pl.BlockSpec((_HID, _VOCAB), lambda i: (0, 0)),
            pl.BlockSpec((1, _VOCAB), lambda i: (0, 0)),
            pl.BlockSpec((_HID, _VVOCAB), lambda i: (0, 0)),
            pl.BlockSpec((1, _VVOCAB), lambda i: (0, 0)),
        ],
        out_specs=[
            pl.BlockSpec((_DB, _T, _VOCAB), lambda i: (i, 0, 0)),
            pl.BlockSpec((_DB, _T, _VVOCAB), lambda i: (i, 0, 0)),
        ],
        out_shape=[
            jax.ShapeDtypeStruct((_B, _T, _VOCAB), jnp.float32),
            jax.ShapeDtypeStruct((_B, _T, _VVOCAB), jnp.float32),
        ],
        compiler_params=pltpu.CompilerParams(
            dimension_semantics=("arbitrary",),
            vmem_limit_bytes=100 * 1024 * 1024,
        ),
    )(hrows, wdT, bd, wvT, bv)


def kernel(x1, x2, encoder, encoder_vel, W_ih, W_hh, b_ih, b_hh, W_dec, b_dec,
           W_dec_vel, b_dec_vel):
    info = plsc.get_sparse_core_info()
    nw = info.num_cores * info.num_subcores
    idx = jnp.concatenate([
        jnp.swapaxes(x1, 0, 1).reshape(-1),
        jnp.swapaxes(x2, 0, 1).reshape(-1),
    ]).astype(jnp.int32)
    idx3 = idx.reshape(nw, -1, _CHUNK)
    table_p = jnp.pad(encoder, ((0, 0), (0, _EMBP - _EMB)))
    rows = _sc_gather_rows(table_p, idx3)
    g = rows.reshape(2, _T, _B, _EMBP)

    bg = (b_ih + b_hh).reshape(1, 4 * _HID)
    bf16 = jnp.bfloat16
    wihT_p = jnp.pad(W_ih.T, ((0, _EMBP - _EMB), (0, 0))).astype(bf16)
    hall, ht, ct = _lstm_call(g, wihT_p, W_hh.T.astype(bf16), bg)

    hpad = jnp.pad(jnp.swapaxes(hall, 0, 1), ((0, 0), (0, _TP - _T), (0, 0)))
    hrows = hpad.reshape(_B * _TP, _HID)

    out, outv = _dec_call(
        hrows, W_dec.T.astype(bf16), b_dec.reshape(1, _VOCAB),
        W_dec_vel.T.astype(bf16), b_dec_vel.reshape(1, _VVOCAB))
    return (out, outv, (ht, ct))
